# Initial kernel scaffold; baseline (speedup 1.0000x reference)
import functools

import jax
import jax.numpy as jnp
from jax import lax
from jax.experimental import pallas as pl
from jax.experimental.pallas import tpu as pltpu
from jax.experimental.pallas import tpu_sc as plsc

EMB_DIM = 32
NUM_WORKERS = 32
CHUNK = 2048


@functools.lru_cache(maxsize=None)
def _build(B: int):
    assert B % (NUM_WORKERS * CHUNK) == 0
    b_per_w = B // NUM_WORKERS
    n_iter = b_per_w // CHUNK
    mesh = plsc.VectorSubcoreMesh(core_axis_name="c", subcore_axis_name="s")

    @functools.partial(
        pl.kernel,
        mesh=mesh,
        out_type=jax.ShapeDtypeStruct((B, EMB_DIM), jnp.float32),
        scratch_types=[
            pltpu.VMEM((CHUNK,), jnp.int32),
            pltpu.VMEM((CHUNK, EMB_DIM), jnp.float32),
            pltpu.SemaphoreType.DMA,
        ],
    )
    def emb_lookup(x_hbm, tbl_hbm, out_hbm, idx_v, rows_v, sem):
        num_cores = 2
        wid = lax.axis_index("s") * num_cores + lax.axis_index("c")
        base = wid * b_per_w

        def body(i, carry):
            off = base + i * CHUNK
            pltpu.sync_copy(x_hbm.at[pl.ds(off, CHUNK)], idx_v)
            pltpu.async_copy(tbl_hbm.at[idx_v], rows_v, sem).wait()
            pltpu.sync_copy(rows_v, out_hbm.at[pl.ds(off, CHUNK)])
            return carry

        lax.fori_loop(0, n_iter, body, 0)

    return emb_lookup


def kernel(x, emb_weight):
    shape = x.shape
    xf = x.reshape(-1).astype(jnp.int32)
    out = _build(xf.shape[0])(xf, emb_weight)
    return out.reshape(*shape, EMB_DIM)


# speedup vs baseline: 4.9473x; 4.9473x over previous
import functools

import jax
import jax.numpy as jnp
from jax import lax
from jax.experimental import pallas as pl
from jax.experimental.pallas import tpu as pltpu
from jax.experimental.pallas import tpu_sc as plsc

EMB_DIM = 32
NUM_WORKERS = 32
CHUNK = 2048


@functools.lru_cache(maxsize=None)
def _build(B: int):
    assert B % (NUM_WORKERS * CHUNK) == 0
    b_per_w = B // NUM_WORKERS
    n_iter = b_per_w // CHUNK
    mesh = plsc.VectorSubcoreMesh(core_axis_name="c", subcore_axis_name="s")

    @functools.partial(
        pl.kernel,
        mesh=mesh,
        out_type=jax.ShapeDtypeStruct((B, EMB_DIM), jnp.float32),
        scratch_types=[
            pltpu.VMEM((CHUNK,), jnp.int32),
            pltpu.VMEM((CHUNK, EMB_DIM), jnp.float32),
            pltpu.SemaphoreType.DMA,
        ],
        compiler_params=pltpu.CompilerParams(use_tc_tiling_on_sc=False),
    )
    def emb_lookup(x_hbm, tbl_hbm, out_hbm, idx_v, rows_v, sem):
        num_cores = 2
        wid = lax.axis_index("s") * num_cores + lax.axis_index("c")
        base = wid * b_per_w

        def body(i, carry):
            off = base + i * CHUNK
            pltpu.sync_copy(x_hbm.at[pl.ds(off, CHUNK)], idx_v)
            pltpu.async_copy(tbl_hbm.at[idx_v], rows_v, sem).wait()
            pltpu.sync_copy(rows_v, out_hbm.at[pl.ds(off, CHUNK)])
            return carry

        lax.fori_loop(0, n_iter, body, 0)

    return emb_lookup


def kernel(x, emb_weight):
    shape = x.shape
    xf = x.reshape(-1).astype(jnp.int32)
    out = _build(xf.shape[0])(xf, emb_weight)
    return out.reshape(*shape, EMB_DIM)


# speedup vs baseline: 5.0396x; 1.0187x over previous
import functools

import jax
import jax.numpy as jnp
from jax import lax
from jax.experimental import pallas as pl
from jax.experimental.pallas import tpu as pltpu
from jax.experimental.pallas import tpu_sc as plsc

EMB_DIM = 32
NUM_WORKERS = 32
CHUNK = 1600
NBUF = 2


@functools.lru_cache(maxsize=None)
def _build(B: int):
    assert B % (NUM_WORKERS * CHUNK * NBUF) == 0
    b_per_w = B // NUM_WORKERS
    n_iter = b_per_w // CHUNK
    n_outer = n_iter // NBUF
    mesh = plsc.VectorSubcoreMesh(core_axis_name="c", subcore_axis_name="s")

    @functools.partial(
        pl.kernel,
        mesh=mesh,
        out_type=jax.ShapeDtypeStruct((B, EMB_DIM), jnp.float32),
        scratch_types=[
            pltpu.VMEM((CHUNK,), jnp.int32),
            pltpu.VMEM((CHUNK,), jnp.int32),
            pltpu.VMEM((CHUNK, EMB_DIM), jnp.float32),
            pltpu.VMEM((CHUNK, EMB_DIM), jnp.float32),
            pltpu.SemaphoreType.DMA,
            pltpu.SemaphoreType.DMA,
            pltpu.SemaphoreType.DMA,
            pltpu.SemaphoreType.DMA,
            pltpu.SemaphoreType.DMA,
            pltpu.SemaphoreType.DMA,
        ],
        compiler_params=pltpu.CompilerParams(use_tc_tiling_on_sc=False),
    )
    def emb_lookup(x_hbm, tbl_hbm, out_hbm, idx0, idx1, rows0, rows1,
                   isem0, isem1, gsem0, gsem1, ssem0, ssem1):
        idx_v = (idx0, idx1)
        rows_v = (rows0, rows1)
        num_cores = 2
        wid = lax.axis_index("s") * num_cores + lax.axis_index("c")
        base = wid * b_per_w
        isem = (isem0, isem1)
        gsem = (gsem0, gsem1)
        ssem = (ssem0, ssem1)

        def load_idx(i, b):
            return pltpu.make_async_copy(
                x_hbm.at[pl.ds(base + i * CHUNK, CHUNK)], idx_v[b], isem[b])

        def gather(b):
            return pltpu.make_async_copy(tbl_hbm.at[idx_v[b]],
                                         rows_v[b], gsem[b])

        def store(i, b):
            return pltpu.make_async_copy(
                rows_v[b], out_hbm.at[pl.ds(base + i * CHUNK, CHUNK)],
                ssem[b])

        for b in range(NBUF):
            load_idx(b, b).start()
        for b in range(NBUF):
            load_idx(b, b).wait()
            g = gather(b)
            g.start()
            g.wait()
            store(b, b).start()
            load_idx(NBUF + b, b).start()

        def body(g, carry):
            for b in range(NBUF):
                i = g * NBUF + b
                load_idx(i, b).wait()
                store(i - NBUF, b).wait()
                gcp = gather(b)
                gcp.start()
                gcp.wait()
                store(i, b).start()
                load_idx(i + NBUF, b).start()
            return carry

        lax.fori_loop(1, n_outer - 1, body, 0, unroll=False)

        for b in range(NBUF):
            i = (n_outer - 1) * NBUF + b
            load_idx(i, b).wait()
            store(i - NBUF, b).wait()
            gcp = gather(b)
            gcp.start()
            gcp.wait()
            store(i, b).start()
        for b in range(NBUF):
            i = (n_outer - 1) * NBUF + b
            store(i, b).wait()

    return emb_lookup


def kernel(x, emb_weight):
    shape = x.shape
    xf = x.reshape(-1).astype(jnp.int32)
    out = _build(xf.shape[0])(xf, emb_weight)
    return out.reshape(*shape, EMB_DIM)
